# baseline (speedup 1.0000x reference)
import functools

import jax
import jax.numpy as jnp
from jax import lax
from jax.experimental import pallas as pl
from jax.experimental.pallas import tpu as pltpu
from jax.experimental.pallas import tpu_sc as plsc

_B, _S, _D = 4, 8192, 2048
_W = _S // 4
_LANES = 16
_CHUNK = 8
_BIG = 1 << 30

_mesh = plsc.VectorSubcoreMesh(core_axis_name="c", subcore_axis_name="s")


def _lane_min(x, iota):
    for sh in (8, 4, 2, 1):
        rot = lax.rem(iota + sh, jnp.full((_LANES,), _LANES, jnp.int32))
        x = jnp.minimum(x, x.at[rot].get(mode="promise_in_bounds"))
    return x[0]


@functools.partial(
    pl.kernel,
    out_type=jax.ShapeDtypeStruct((_B, _D), jnp.float32),
    mesh=_mesh,
    scratch_types=[
        pltpu.VMEM((_W,), jnp.int32),
    ],
)
def _extract_eos(tokens_hbm, words_hbm, out_hbm, words_v):
    num_c = lax.axis_size("c")
    wid = lax.axis_index("s") * num_c + lax.axis_index("c")

    @pl.when(wid < _B)
    def _():
        b = wid
        pltpu.sync_copy(words_hbm.at[b], words_v)
        iota = lax.iota(jnp.int32, _LANES)
        n_iters = _W // (_LANES * _CHUNK)

        def body(j, wcand):
            base = j * (_LANES * _CHUNK)
            for k in range(_CHUNK):
                off = base + k * _LANES
                v = words_v[pl.ds(off, _LANES)]
                wcand = jnp.minimum(
                    wcand, jnp.where(v != 0, iota + off, _BIG)
                )
            return wcand

        wcand0 = jnp.full((_LANES,), _BIG, jnp.int32)
        wcand = lax.fori_loop(0, n_iters, body, wcand0)
        w = _lane_min(wcand, iota)

        cbase = jnp.minimum(w, jnp.int32(_W - 1)) // _LANES * _LANES
        v = words_v[pl.ds(pl.multiple_of(cbase, _LANES), _LANES)]
        b0 = (v & 0x000000FF) != 0
        b1 = (v & 0x0000FF00) != 0
        b2 = (v & 0x00FF0000) != 0
        sub = jnp.where(b0, 0, jnp.where(b1, 1, jnp.where(b2, 2, 3)))
        pos = (iota + cbase) * 4 + sub.astype(jnp.int32)
        m = _lane_min(jnp.where(v != 0, pos, _BIG), iota)
        idx = jnp.where(m < _BIG, m, jnp.int32(0))

        pltpu.sync_copy(tokens_hbm.at[b, idx], out_hbm.at[b])


def kernel(tokens, eos_token_mask):
    words = lax.bitcast_convert_type(
        eos_token_mask.reshape(_B, _W, 4).view(jnp.uint8), jnp.int32
    )
    return _extract_eos(tokens, words)


# speedup vs baseline: 1.1638x; 1.1638x over previous
import functools

import jax
import jax.numpy as jnp
from jax import lax
from jax.experimental import pallas as pl
from jax.experimental.pallas import tpu as pltpu
from jax.experimental.pallas import tpu_sc as plsc

_B, _S, _D = 4, 8192, 2048
_LANES = 16
_CHUNK = 16
_BIG = 1 << 30

_mesh = plsc.VectorSubcoreMesh(core_axis_name="c", subcore_axis_name="s")


def _lane_min(x, iota):
    for sh in (8, 4, 2, 1):
        rot = lax.rem(iota + sh, jnp.full((_LANES,), _LANES, jnp.int32))
        x = jnp.minimum(x, x.at[rot].get(mode="promise_in_bounds"))
    return x[0]


@functools.partial(
    pl.kernel,
    out_type=jax.ShapeDtypeStruct((_B, _D), jnp.float32),
    mesh=_mesh,
    scratch_types=[
        pltpu.VMEM((_S,), jnp.int32),
    ],
)
def _extract_eos(tokens_hbm, mask_hbm, out_hbm, words_v):
    num_c = lax.axis_size("c")
    wid = lax.axis_index("s") * num_c + lax.axis_index("c")

    @pl.when(wid < _B)
    def _():
        b = wid
        words_hbm = mask_hbm.bitcast(jnp.int32)
        pltpu.sync_copy(words_hbm.at[0], words_v)
        iota = lax.iota(jnp.int32, _LANES)
        bmask = jnp.full((_LANES,), 0xFF, jnp.int32) << jnp.full(
            (_LANES,), 8 * b, jnp.int32
        )
        n_iters = _S // (_LANES * _CHUNK)

        def body(j, cand):
            base = j * (_LANES * _CHUNK)
            for k in range(_CHUNK):
                off = base + k * _LANES
                v = words_v[pl.ds(off, _LANES)]
                cand = jnp.minimum(
                    cand, jnp.where((v & bmask) != 0, iota + off, _BIG)
                )
            return cand

        cand0 = jnp.full((_LANES,), _BIG, jnp.int32)
        cand = lax.fori_loop(0, n_iters, body, cand0)
        m = _lane_min(cand, iota)
        idx = jnp.where(m < _BIG, m, jnp.int32(0))

        pltpu.sync_copy(tokens_hbm.at[b, idx], out_hbm.at[b])


def kernel(tokens, eos_token_mask):
    return _extract_eos(tokens, eos_token_mask.view(jnp.uint8))


# speedup vs baseline: 1.2011x; 1.0321x over previous
import functools

import jax
import jax.numpy as jnp
from jax import lax
from jax.experimental import pallas as pl
from jax.experimental.pallas import tpu as pltpu
from jax.experimental.pallas import tpu_sc as plsc

_B, _S, _D = 4, 8192, 2048
_LANES = 16
_NSEG = 8
_SEG = _S // _NSEG
_CHUNK = 16
_BIG = 1 << 30

_mesh = plsc.VectorSubcoreMesh(core_axis_name="c", subcore_axis_name="s")


def _lane_min(x, iota):
    for sh in (8, 4, 2, 1):
        rot = lax.rem(iota + sh, jnp.full((_LANES,), _LANES, jnp.int32))
        x = jnp.minimum(x, x.at[rot].get(mode="promise_in_bounds"))
    return x[0]


@functools.partial(
    pl.kernel,
    out_type=jax.ShapeDtypeStruct((_B, _D), jnp.float32),
    mesh=_mesh,
    scratch_types=[
        pltpu.VMEM((_SEG,), jnp.int32),
        pltpu.VMEM((_LANES,), jnp.int32),
        pltpu.VMEM((_NSEG * _LANES,), jnp.int32),
        pltpu.VMEM_SHARED((16 * _LANES,), jnp.int32),
    ],
)
def _extract_eos(tokens_hbm, mask_hbm, out_hbm, words_v, cand_v, comb_v, shared):
    c = lax.axis_index("c")
    s = lax.axis_index("s")
    b = 2 * c + s // _NSEG
    g = s % _NSEG

    words_hbm = mask_hbm.bitcast(jnp.int32)
    seg0 = pl.multiple_of(g * _SEG, _SEG)
    pltpu.sync_copy(words_hbm.at[0, pl.ds(seg0, _SEG)], words_v)

    iota = lax.iota(jnp.int32, _LANES)
    bmask = jnp.full((_LANES,), 0xFF, jnp.int32) << jnp.full(
        (_LANES,), 8 * b, jnp.int32
    )
    base_pos = g * _SEG
    n_iters = _SEG // (_LANES * _CHUNK)

    def body(j, cand):
        base = j * (_LANES * _CHUNK)
        for k in range(_CHUNK):
            off = base + k * _LANES
            v = words_v[pl.ds(off, _LANES)]
            cand = jnp.minimum(
                cand, jnp.where((v & bmask) != 0, iota + (base_pos + off), _BIG)
            )
        return cand

    cand0 = jnp.full((_LANES,), _BIG, jnp.int32)
    cand = lax.fori_loop(0, n_iters, body, cand0)
    cand_v[...] = cand
    pltpu.sync_copy(
        cand_v, shared.at[pl.ds(pl.multiple_of(s * _LANES, _LANES), _LANES)]
    )
    plsc.subcore_barrier()

    @pl.when(g == 0)
    def _():
        base_slot = pl.multiple_of((s // _NSEG) * (_NSEG * _LANES), _NSEG * _LANES)
        pltpu.sync_copy(
            shared.at[pl.ds(base_slot, _NSEG * _LANES)], comb_v
        )
        acc = comb_v[pl.ds(0, _LANES)]
        for r in range(1, _NSEG):
            acc = jnp.minimum(acc, comb_v[pl.ds(r * _LANES, _LANES)])
        m = _lane_min(acc, iota)
        idx = jnp.where(m < _BIG, m, jnp.int32(0))
        pltpu.sync_copy(tokens_hbm.at[b, idx], out_hbm.at[b])


def kernel(tokens, eos_token_mask):
    return _extract_eos(tokens, eos_token_mask.view(jnp.uint8))


# speedup vs baseline: 1.2146x; 1.0113x over previous
import functools

import jax
import jax.numpy as jnp
from jax import lax
from jax.experimental import pallas as pl
from jax.experimental.pallas import tpu as pltpu
from jax.experimental.pallas import tpu_sc as plsc

_B, _S, _D = 4, 8192, 2048
_LANES = 16
_NSEG = 8
_CHUNK = 8
_BLK = _LANES * _CHUNK
_HEAD = 2 * _BLK
_SEG = 9 * _BLK
_LAST = _S - _SEG
_BIG = 1 << 30

_mesh = plsc.VectorSubcoreMesh(core_axis_name="c", subcore_axis_name="s")


def _lane_min(x, iota):
    for sh in (8, 4, 2, 1):
        rot = lax.rem(iota + sh, jnp.full((_LANES,), _LANES, jnp.int32))
        x = jnp.minimum(x, x.at[rot].get(mode="promise_in_bounds"))
    return x[0]


@functools.partial(
    pl.kernel,
    out_type=jax.ShapeDtypeStruct((_B, _D), jnp.float32),
    mesh=_mesh,
    scratch_types=[
        pltpu.VMEM((_SEG,), jnp.int32),
        pltpu.VMEM((_LANES,), jnp.int32),
        pltpu.VMEM((_NSEG * _LANES,), jnp.int32),
        pltpu.VMEM_SHARED((16 * _LANES,), jnp.int32),
    ],
)
def _extract_eos(tokens_hbm, mask_hbm, out_hbm, words_v, cand_v, comb_v, shared):
    c = lax.axis_index("c")
    s = lax.axis_index("s")
    b = 2 * c + s // _NSEG
    g = s % _NSEG

    words_hbm = mask_hbm.bitcast(jnp.int32)
    base_w = jnp.where(
        g == 0,
        jnp.int32(0),
        jnp.minimum(_HEAD + (g - 1) * _SEG, jnp.int32(_LAST)),
    )
    base_w = pl.multiple_of(base_w, 8)
    pltpu.sync_copy(words_hbm.at[0, pl.ds(base_w, _SEG)], words_v)

    iota = lax.iota(jnp.int32, _LANES)
    bmask = jnp.full((_LANES,), 0xFF, jnp.int32) << jnp.full(
        (_LANES,), 8 * b, jnp.int32
    )
    n_iters = jnp.where(g == 0, _HEAD // _BLK, _SEG // _BLK)

    def body(j, cand):
        base = j * _BLK
        for k in range(_CHUNK):
            off = base + k * _LANES
            v = words_v[pl.ds(off, _LANES)]
            cand = jnp.minimum(
                cand, jnp.where((v & bmask) != 0, iota + (base_w + off), _BIG)
            )
        return cand

    cand0 = jnp.full((_LANES,), _BIG, jnp.int32)
    cand = lax.fori_loop(0, n_iters, body, cand0)
    cand_v[...] = cand
    pltpu.sync_copy(
        cand_v, shared.at[pl.ds(pl.multiple_of(s * _LANES, _LANES), _LANES)]
    )
    m0 = _lane_min(cand, iota)

    @pl.when(jnp.logical_and(g == 0, m0 < _BIG))
    def _():
        pltpu.sync_copy(tokens_hbm.at[b, m0], out_hbm.at[b])

    plsc.subcore_barrier()

    @pl.when(jnp.logical_and(g == 0, m0 >= _BIG))
    def _():
        base_slot = pl.multiple_of((s // _NSEG) * (_NSEG * _LANES), _NSEG * _LANES)
        pltpu.sync_copy(
            shared.at[pl.ds(base_slot, _NSEG * _LANES)], comb_v
        )
        acc = comb_v[pl.ds(0, _LANES)]
        for r in range(1, _NSEG):
            acc = jnp.minimum(acc, comb_v[pl.ds(r * _LANES, _LANES)])
        m = _lane_min(acc, iota)
        idx = jnp.where(m < _BIG, m, jnp.int32(0))
        pltpu.sync_copy(tokens_hbm.at[b, idx], out_hbm.at[b])


def kernel(tokens, eos_token_mask):
    return _extract_eos(tokens, eos_token_mask.view(jnp.uint8))
